# baseline (speedup 1.0000x reference)
import dataclasses

import jax
import jax.numpy as jnp
from jax import lax
from jax.experimental import pallas as pl
from jax.experimental.pallas import tpu as pltpu
from jax.experimental.pallas import tpu_sc as plsc

D = 128
K = 20
R = K + 1
NC = 2
NS = 16
NW = NC * NS
L = 16
CHUNK = 16
GSPLIT = 112


def _build_sc_call(B):
    spw = B // NW
    n_chunks = spw // CHUNK
    u_rows = CHUNK * R

    mesh = plsc.VectorSubcoreMesh(core_axis_name="c", subcore_axis_name="s")

    def body(cw_hbm, uidx_hbm, cemb_hbm, uemb_hbm, out_hbm,
             cidx_v, uidx_v, cbuf, ubuf, stage, obuf, sem0, sem1):
        wid = lax.axis_index("s") * NC + lax.axis_index("c")
        sbase = wid * spw
        pltpu.sync_copy(cw_hbm.at[pl.ds(sbase, spw)], cidx_v)
        pltpu.sync_copy(uidx_hbm.at[pl.ds(sbase * R, spw * R)], uidx_v)

        lanes = lax.iota(jnp.int32, L)
        scat_base = lanes * u_rows
        sems = (sem0, sem1)

        def fire(g, b):
            pltpu.async_copy(
                cemb_hbm.at[cidx_v.at[pl.ds(g * CHUNK, CHUNK)]],
                cbuf.at[b], sems[b])
            for p in range(u_rows // GSPLIT):
                pltpu.async_copy(
                    uemb_hbm.at[uidx_v.at[pl.ds(g * u_rows + p * GSPLIT,
                                                GSPLIT)]],
                    ubuf.at[b, pl.ds(p * GSPLIT, GSPLIT)], sems[b])

        def drain(b):
            pltpu.make_async_copy(
                cemb_hbm.at[pl.ds(0, CHUNK)], cbuf.at[b], sems[b]).wait()
            pltpu.make_async_copy(
                uemb_hbm.at[pl.ds(0, u_rows)], ubuf.at[b], sems[b]).wait()

        def compute(g, b):
            @pl.loop(0, CHUNK)
            def _sample(i):
                v = [cbuf[b, i, pl.ds(t * L, L)] for t in range(D // L)]

                @pl.loop(0, R)
                def _dot(j):
                    row = i * R + j
                    idx = scat_base + row
                    plsc.store_scatter(
                        stage, [idx], v[0] * ubuf[b, row, pl.ds(0, L)])
                    for t in range(1, D // L):
                        plsc.addupdate_scatter(
                            stage, [idx], v[t] * ubuf[b, row, pl.ds(t * L, L)])

            @pl.loop(0, u_rows // L)
            def _reduce(q):
                s = stage[pl.ds(q * L, L)]
                for t in range(1, L):
                    s = s + stage[pl.ds(t * u_rows + q * L, L)]
                s = jnp.minimum(jnp.maximum(s, -10.0), 10.0)
                obuf[pl.ds(g * u_rows + q * L, L)] = s

        fire(0, 0)

        @pl.loop(0, n_chunks, step=2)
        def _pair(g):
            fire(g + 1, 1)
            drain(0)
            compute(g, 0)

            @pl.when(g + 2 < n_chunks)
            def _():
                fire(g + 2, 0)

            drain(1)
            compute(g + 1, 1)

        pltpu.sync_copy(obuf, out_hbm.at[pl.ds(sbase * R, spw * R)])

    cp = pltpu.CompilerParams()
    if "needs_layout_passes" in pltpu.CompilerParams.__dataclass_fields__:
        cp = dataclasses.replace(cp, needs_layout_passes=False)
    return pl.kernel(
        body,
        out_type=jax.ShapeDtypeStruct((B * R,), jnp.float32),
        mesh=mesh,
        compiler_params=cp,
        scratch_types=[
            pltpu.VMEM((spw,), jnp.int32),
            pltpu.VMEM((spw * R,), jnp.int32),
            pltpu.VMEM((2, CHUNK, D), jnp.float32),
            pltpu.VMEM((2, u_rows, D), jnp.float32),
            pltpu.VMEM((L * u_rows,), jnp.float32),
            pltpu.VMEM((spw * R,), jnp.float32),
            pltpu.SemaphoreType.DMA,
            pltpu.SemaphoreType.DMA,
        ],
    )


def kernel(center_words, context_words, negative_samples, center_emb,
           context_emb):
    B = center_words.shape[0]
    cw = center_words.astype(jnp.int32)
    u_idx = jnp.concatenate(
        [context_words.astype(jnp.int32)[:, None],
         negative_samples.astype(jnp.int32)], axis=1).reshape(B * R)
    out = _build_sc_call(B)(cw, u_idx, center_emb, context_emb)
    out = out.reshape(B, R)
    return out[:, 0], out[:, 1:]


# speedup vs baseline: 5.2359x; 5.2359x over previous
import dataclasses

import jax
import jax.numpy as jnp
from jax import lax
from jax.experimental import pallas as pl
from jax.experimental.pallas import tpu as pltpu
from jax.experimental.pallas import tpu_sc as plsc

D = 128
K = 20
R = K + 1
NC = 2
NS = 16
NW = NC * NS
L = 16
CHUNK = 16
GSPLIT = 112


def _build_sc_call(B):
    spw = B // NW
    n_chunks = spw // CHUNK
    u_rows = CHUNK * R

    mesh = plsc.VectorSubcoreMesh(core_axis_name="c", subcore_axis_name="s")

    def body(cw_hbm, uidx_hbm, cemb_hbm, uemb_hbm, out_hbm,
             cidx_v, uidx_v, cbuf, ubuf, stage, obuf, sem0, sem1):
        wid = lax.axis_index("s") * NC + lax.axis_index("c")
        sbase = wid * spw
        pltpu.sync_copy(cw_hbm.at[pl.ds(sbase, spw)], cidx_v)
        pltpu.sync_copy(uidx_hbm.at[pl.ds(sbase * R, spw * R)], uidx_v)

        lanes = lax.iota(jnp.int32, L)
        scat_base = lanes * u_rows
        sems = (sem0, sem1)

        def fire(g, b):
            pltpu.async_copy(
                cemb_hbm.at[cidx_v.at[pl.ds(g * CHUNK, CHUNK)]],
                cbuf.at[b], sems[b])
            for p in range(u_rows // GSPLIT):
                pltpu.async_copy(
                    uemb_hbm.at[uidx_v.at[pl.ds(g * u_rows + p * GSPLIT,
                                                GSPLIT)]],
                    ubuf.at[b, pl.ds(p * GSPLIT, GSPLIT)], sems[b])

        def drain(b):
            pltpu.make_async_copy(
                cemb_hbm.at[pl.ds(0, CHUNK)], cbuf.at[b], sems[b]).wait()
            pltpu.make_async_copy(
                uemb_hbm.at[pl.ds(0, u_rows)], ubuf.at[b], sems[b]).wait()

        def compute(g, b):
            @pl.loop(0, CHUNK)
            def _sample(i):
                v = [cbuf[b, i, pl.ds(t * L, L)] for t in range(D // L)]

                @plsc.parallel_loop(0, R, unroll=3)
                def _dot(j):
                    row = i * R + j
                    p = [v[t] * ubuf[b, row, pl.ds(t * L, L)]
                         for t in range(D // L)]
                    while len(p) > 1:
                        p = [p[t] + p[t + 1] for t in range(0, len(p), 2)]
                    plsc.store_scatter(stage, [scat_base + row], p[0])

            @pl.loop(0, u_rows // L)
            def _reduce(q):
                s = stage[pl.ds(q * L, L)]
                for t in range(1, L):
                    s = s + stage[pl.ds(t * u_rows + q * L, L)]
                s = jnp.minimum(jnp.maximum(s, -10.0), 10.0)
                obuf[pl.ds(g * u_rows + q * L, L)] = s

        fire(0, 0)

        @pl.loop(0, n_chunks, step=2)
        def _pair(g):
            fire(g + 1, 1)
            drain(0)
            compute(g, 0)

            @pl.when(g + 2 < n_chunks)
            def _():
                fire(g + 2, 0)

            drain(1)
            compute(g + 1, 1)

        pltpu.sync_copy(obuf, out_hbm.at[pl.ds(sbase * R, spw * R)])

    cp = pltpu.CompilerParams()
    if "needs_layout_passes" in pltpu.CompilerParams.__dataclass_fields__:
        cp = dataclasses.replace(cp, needs_layout_passes=False)
    return pl.kernel(
        body,
        out_type=jax.ShapeDtypeStruct((B * R,), jnp.float32),
        mesh=mesh,
        compiler_params=cp,
        scratch_types=[
            pltpu.VMEM((spw,), jnp.int32),
            pltpu.VMEM((spw * R,), jnp.int32),
            pltpu.VMEM((2, CHUNK, D), jnp.float32),
            pltpu.VMEM((2, u_rows, D), jnp.float32),
            pltpu.VMEM((L * u_rows,), jnp.float32),
            pltpu.VMEM((spw * R,), jnp.float32),
            pltpu.SemaphoreType.DMA,
            pltpu.SemaphoreType.DMA,
        ],
    )


def kernel(center_words, context_words, negative_samples, center_emb,
           context_emb):
    B = center_words.shape[0]
    cw = center_words.astype(jnp.int32)
    u_idx = jnp.concatenate(
        [context_words.astype(jnp.int32)[:, None],
         negative_samples.astype(jnp.int32)], axis=1).reshape(B * R)
    out = _build_sc_call(B)(cw, u_idx, center_emb, context_emb)
    out = out.reshape(B, R)
    return out[:, 0], out[:, 1:]


# speedup vs baseline: 5.4006x; 1.0315x over previous
import dataclasses

import jax
import jax.numpy as jnp
from jax import lax
from jax.experimental import pallas as pl
from jax.experimental.pallas import tpu as pltpu
from jax.experimental.pallas import tpu_sc as plsc

D = 128
K = 20
R = K + 1
NC = 2
NS = 16
NW = NC * NS
L = 16
CHUNK = 16
GSPLIT = 112


def _build_sc_call(B):
    spw = B // NW
    n_chunks = spw // CHUNK
    u_rows = CHUNK * R

    mesh = plsc.VectorSubcoreMesh(core_axis_name="c", subcore_axis_name="s")

    def body(cw_hbm, uidx_hbm, cemb_hbm, uemb_hbm, out_hbm,
             cidx_v, uidx_v, cbuf, ubuf, stage, obuf, sem0, sem1):
        wid = lax.axis_index("s") * NC + lax.axis_index("c")
        sbase = wid * spw
        pltpu.sync_copy(cw_hbm.at[pl.ds(sbase, spw)], cidx_v)
        pltpu.sync_copy(uidx_hbm.at[pl.ds(sbase * R, spw * R)], uidx_v)

        lanes = lax.iota(jnp.int32, L)
        scat_base = lanes * u_rows
        sems = (sem0, sem1)

        def fire(g, b):
            pltpu.async_copy(
                cemb_hbm.at[cidx_v.at[pl.ds(g * CHUNK, CHUNK)]],
                cbuf.at[b], sems[b])
            for p in range(u_rows // GSPLIT):
                pltpu.async_copy(
                    uemb_hbm.at[uidx_v.at[pl.ds(g * u_rows + p * GSPLIT,
                                                GSPLIT)]],
                    ubuf.at[b, pl.ds(p * GSPLIT, GSPLIT)], sems[b])

        def drain(b):
            pltpu.make_async_copy(
                cemb_hbm.at[pl.ds(0, CHUNK)], cbuf.at[b], sems[b]).wait()
            pltpu.make_async_copy(
                uemb_hbm.at[pl.ds(0, u_rows)], ubuf.at[b], sems[b]).wait()

        def compute(g, b):
            @pl.loop(0, CHUNK)
            def _sample(i):
                v = [cbuf[b, i, pl.ds(t * L, L)] for t in range(D // L)]

                @plsc.parallel_loop(0, R, unroll=3)
                def _dot(j):
                    row = i * R + j
                    p = [v[t] * ubuf[b, row, pl.ds(t * L, L)]
                         for t in range(D // L)]
                    while len(p) > 1:
                        p = [p[t] + p[t + 1] for t in range(0, len(p), 2)]
                    plsc.store_scatter(stage, [scat_base + row], p[0])

            @plsc.parallel_loop(0, u_rows // L, unroll=2)
            def _reduce(q):
                s = stage[pl.ds(q * L, L)]
                for t in range(1, L):
                    s = s + stage[pl.ds(t * u_rows + q * L, L)]
                s = jnp.minimum(jnp.maximum(s, -10.0), 10.0)
                obuf[pl.ds(g * u_rows + q * L, L)] = s

        fire(0, 0)

        @pl.loop(0, n_chunks, step=2)
        def _pair(g):
            fire(g + 1, 1)
            drain(0)
            compute(g, 0)

            @pl.when(g + 2 < n_chunks)
            def _():
                fire(g + 2, 0)

            drain(1)
            compute(g + 1, 1)

        pltpu.sync_copy(obuf, out_hbm.at[pl.ds(sbase * R, spw * R)])

    cp = pltpu.CompilerParams()
    if "needs_layout_passes" in pltpu.CompilerParams.__dataclass_fields__:
        cp = dataclasses.replace(cp, needs_layout_passes=False)
    return pl.kernel(
        body,
        out_type=jax.ShapeDtypeStruct((B * R,), jnp.float32),
        mesh=mesh,
        compiler_params=cp,
        scratch_types=[
            pltpu.VMEM((spw,), jnp.int32),
            pltpu.VMEM((spw * R,), jnp.int32),
            pltpu.VMEM((2, CHUNK, D), jnp.float32),
            pltpu.VMEM((2, u_rows, D), jnp.float32),
            pltpu.VMEM((L * u_rows,), jnp.float32),
            pltpu.VMEM((spw * R,), jnp.float32),
            pltpu.SemaphoreType.DMA,
            pltpu.SemaphoreType.DMA,
        ],
    )


def kernel(center_words, context_words, negative_samples, center_emb,
           context_emb):
    B = center_words.shape[0]
    cw = center_words.astype(jnp.int32)
    u_idx = jnp.concatenate(
        [context_words.astype(jnp.int32)[:, None],
         negative_samples.astype(jnp.int32)], axis=1).reshape(B * R)
    out = _build_sc_call(B)(cw, u_idx, center_emb, context_emb)
    out = out.reshape(B, R)
    return out[:, 0], out[:, 1:]


# speedup vs baseline: 5.4318x; 1.0058x over previous
import dataclasses

import jax
import jax.numpy as jnp
from jax import lax
from jax.experimental import pallas as pl
from jax.experimental.pallas import tpu as pltpu
from jax.experimental.pallas import tpu_sc as plsc

D = 128
K = 20
R = K + 1
NC = 2
NS = 16
NW = NC * NS
L = 16
CHUNK = 16
GSPLIT = 112


def _build_sc_call(B):
    spw = B // NW
    n_chunks = spw // CHUNK
    u_rows = CHUNK * R

    mesh = plsc.VectorSubcoreMesh(core_axis_name="c", subcore_axis_name="s")

    def body(cw_hbm, uidx_hbm, cemb_hbm, uemb_hbm, out_hbm,
             cidx_v, uidx_v, cbuf, ubuf, stage, obuf, sem0, sem1):
        wid = lax.axis_index("s") * NC + lax.axis_index("c")
        sbase = wid * spw
        pltpu.sync_copy(cw_hbm.at[pl.ds(sbase, spw)], cidx_v)
        pltpu.sync_copy(uidx_hbm.at[pl.ds(sbase * R, spw * R)], uidx_v)

        lanes = lax.iota(jnp.int32, L)
        scat_base = lanes * u_rows
        sems = (sem0, sem1)

        def fire(g, b):
            pltpu.async_copy(
                cemb_hbm.at[cidx_v.at[pl.ds(g * CHUNK, CHUNK)]],
                cbuf.at[b], sems[b])
            for p in range(u_rows // GSPLIT):
                pltpu.async_copy(
                    uemb_hbm.at[uidx_v.at[pl.ds(g * u_rows + p * GSPLIT,
                                                GSPLIT)]],
                    ubuf.at[b, pl.ds(p * GSPLIT, GSPLIT)], sems[b])

        def drain(b):
            pltpu.make_async_copy(
                cemb_hbm.at[pl.ds(0, CHUNK)], cbuf.at[b], sems[b]).wait()
            pltpu.make_async_copy(
                uemb_hbm.at[pl.ds(0, u_rows)], ubuf.at[b], sems[b]).wait()

        def compute(g, b):
            @plsc.parallel_loop(0, CHUNK, unroll=2)
            def _sample(i):
                v = [cbuf[b, i, pl.ds(t * L, L)] for t in range(D // L)]

                @plsc.parallel_loop(0, R, unroll=3)
                def _dot(j):
                    row = i * R + j
                    p = [v[t] * ubuf[b, row, pl.ds(t * L, L)]
                         for t in range(D // L)]
                    while len(p) > 1:
                        p = [p[t] + p[t + 1] for t in range(0, len(p), 2)]
                    plsc.store_scatter(stage, [scat_base + row], p[0])

            @plsc.parallel_loop(0, u_rows // L, unroll=2)
            def _reduce(q):
                s = stage[pl.ds(q * L, L)]
                for t in range(1, L):
                    s = s + stage[pl.ds(t * u_rows + q * L, L)]
                s = jnp.minimum(jnp.maximum(s, -10.0), 10.0)
                obuf[pl.ds(g * u_rows + q * L, L)] = s

        fire(0, 0)

        @pl.loop(0, n_chunks, step=2)
        def _pair(g):
            fire(g + 1, 1)
            drain(0)
            compute(g, 0)

            @pl.when(g + 2 < n_chunks)
            def _():
                fire(g + 2, 0)

            drain(1)
            compute(g + 1, 1)

        pltpu.sync_copy(obuf, out_hbm.at[pl.ds(sbase * R, spw * R)])

    cp = pltpu.CompilerParams()
    if "needs_layout_passes" in pltpu.CompilerParams.__dataclass_fields__:
        cp = dataclasses.replace(cp, needs_layout_passes=False)
    return pl.kernel(
        body,
        out_type=jax.ShapeDtypeStruct((B * R,), jnp.float32),
        mesh=mesh,
        compiler_params=cp,
        scratch_types=[
            pltpu.VMEM((spw,), jnp.int32),
            pltpu.VMEM((spw * R,), jnp.int32),
            pltpu.VMEM((2, CHUNK, D), jnp.float32),
            pltpu.VMEM((2, u_rows, D), jnp.float32),
            pltpu.VMEM((L * u_rows,), jnp.float32),
            pltpu.VMEM((spw * R,), jnp.float32),
            pltpu.SemaphoreType.DMA,
            pltpu.SemaphoreType.DMA,
        ],
    )


def kernel(center_words, context_words, negative_samples, center_emb,
           context_emb):
    B = center_words.shape[0]
    cw = center_words.astype(jnp.int32)
    u_idx = jnp.concatenate(
        [context_words.astype(jnp.int32)[:, None],
         negative_samples.astype(jnp.int32)], axis=1).reshape(B * R)
    out = _build_sc_call(B)(cw, u_idx, center_emb, context_emb)
    out = out.reshape(B, R)
    return out[:, 0], out[:, 1:]
